# baseline (speedup 1.0000x reference)
import functools

import jax
import jax.numpy as jnp
from jax import lax
from jax.experimental import pallas as pl
from jax.experimental.pallas import tpu as pltpu
from jax.experimental.pallas import tpu_sc as plsc

_BINS = 8
_DREF = 5
_DVAL = 16
_G = _BINS ** _DREF
_NCORN = 1 << _DREF
_STRIDES = [_BINS ** (_DREF - 1 - d) for d in range(_DREF)]

_NTILES = 32
_PPT = 2048
_SUB = _PPT // 128
_HP = _PPT // 2
_SUBH = _SUB // 2

_SC_PARAMS = pltpu.CompilerParams(needs_layout_passes=False,
                                  use_tc_tiling_on_sc=False)


def _corners_body(ref_ref, idx_ref, w_ref):
    r = ref_ref[...]
    t0, t1, a0, a1 = [], [], [], []
    for d in range(_DREF):
        scaled = r[d] * (_BINS - 1)
        lo = jnp.floor(scaled)
        frac = scaled - lo
        lo_i = lo.astype(jnp.int32)
        t0.append(1.0 - frac)
        t1.append(frac)
        a0.append(jnp.clip(lo_i, 0, _BINS - 1) * _STRIDES[d])
        a1.append(jnp.clip(lo_i + 1, 0, _BINS - 1) * _STRIDES[d])
    for c in range(_NCORN):
        w = jnp.ones((16, 128), jnp.float32)
        idx = jnp.zeros((16, 128), jnp.int32)
        for d in range(_DREF):
            if (c >> d) & 1:
                w = w * t1[d]
                idx = idx + a1[d]
            else:
                w = w * t0[d]
                idx = idx + a0[d]
        idx_ref[c] = idx
        w_ref[c] = w


def _corners(ref3):
    nr = ref3.shape[1]
    nb = nr // 16
    return pl.pallas_call(
        _corners_body,
        grid=(nb,),
        in_specs=[pl.BlockSpec((_DREF, 16, 128), lambda i: (0, i, 0))],
        out_specs=[pl.BlockSpec((_NCORN, 16, 128), lambda i: (0, i, 0)),
                   pl.BlockSpec((_NCORN, 16, 128), lambda i: (0, i, 0))],
        out_shape=[jax.ShapeDtypeStruct((_NCORN, nr, 128), jnp.int32),
                   jax.ShapeDtypeStruct((_NCORN, nr, 128), jnp.float32)],
    )(ref3)


def _blur_body(g_ref, out_ref):
    g = (g_ref[0] + g_ref[1]).T
    lane = lax.broadcasted_iota(jnp.int32, (_DVAL, _G), 1)
    for d in range(_DREF):
        s = _STRIDES[d]
        coord = (lane // s) % _BINS
        zs = jnp.zeros((_DVAL, s), jnp.float32)
        gl = jnp.concatenate([zs, g[:, : _G - s]], axis=1)
        gr = jnp.concatenate([g[:, s:], zs], axis=1)
        g = 0.5 * g + 0.25 * (jnp.where(coord > 0, gl, 0.0) +
                              jnp.where(coord < _BINS - 1, gr, 0.0))
    out_ref[...] = g.T


def _blur(pg):
    return pl.pallas_call(
        _blur_body,
        out_shape=jax.ShapeDtypeStruct((_G, _DVAL), jnp.float32),
    )(pg)


def _bcast_lane(vec, c):
    return jnp.take_along_axis(vec, jnp.full((16,), c, jnp.int32), axis=0)


def _splat(U, idx3, w3):
    mesh = plsc.VectorSubcoreMesh(core_axis_name="c", subcore_axis_name="s")

    @functools.partial(
        pl.kernel,
        mesh=mesh,
        compiler_params=_SC_PARAMS,
        out_type=jax.ShapeDtypeStruct((2, _G, _DVAL), jnp.float32),
        scratch_types=[
            pltpu.VMEM((_SUBH, 128), jnp.int32),
            pltpu.VMEM((_SUBH, 128), jnp.int32),
            pltpu.VMEM((_SUBH, 128), jnp.float32),
            pltpu.VMEM((_PPT, _DVAL), jnp.float32),
            pltpu.VMEM((_HP, _DVAL), jnp.float32),
            pltpu.VMEM((_HP, _DVAL), jnp.float32),
            pltpu.VMEM_SHARED((_G, _DVAL), jnp.float32),
            pltpu.SemaphoreType.DMA,
            pltpu.SemaphoreType.DMA,
        ],
    )
    def k(u_hbm, idx_hbm, w_hbm, out_hbm,
          idxA, idxB, wb, ustage, rowsA, rowsB, sgrid, semA, semB):
        cid = lax.axis_index("c")
        sid = lax.axis_index("s")
        wid = sid * 2 + cid
        rpt = _G // 16
        slab = pl.multiple_of(wid * _SUB, _SUB)

        def zrow(i, c2):
            ustage[i, :] = jnp.zeros((16,), jnp.float32)
            return c2
        lax.fori_loop(0, _PPT, zrow, 0)
        pltpu.sync_copy(ustage, sgrid.at[pl.ds(sid * rpt, rpt)])
        plsc.subcore_barrier()
        pltpu.sync_copy(u_hbm.at[pl.ds(wid * _PPT, _PPT)], ustage)

        def compute(k, rows):
            c = k // 2
            ho = k % 2
            pltpu.sync_copy(w_hbm.at[c, pl.ds(slab + ho * _SUBH, _SUBH)], wb)

            def grp(g, c2):
                r16 = g // 8
                col = (g % 8) * 16
                wv = wb[r16, pl.ds(col, 16)]
                base = r16 * 128 + col
                ub = ho * _HP + base
                for t in range(16):
                    rows[base + t, :] = _bcast_lane(wv, t) * ustage[ub + t, :]
                return c2
            lax.fori_loop(0, _HP // 16, grp, 0)

        def fire(rows, idxb, sem):
            for b in range(_SUBH):
                pltpu.async_copy(rows.at[pl.ds(b * 128, 128)],
                                 sgrid.at[idxb.at[b]], sem, add=True)

        def drain(rows, sem):
            pltpu.make_async_copy(u_hbm.at[pl.ds(0, _HP)], rows, sem).wait()

        def loadidx(k, idxb):
            c = k // 2
            ho = k % 2
            pltpu.sync_copy(idx_hbm.at[c, pl.ds(slab + ho * _SUBH, _SUBH)], idxb)

        def pair(i, carry):
            k0 = 2 * i
            k1 = 2 * i + 1
            loadidx(k0, idxA)
            compute(k0, rowsA)

            @pl.when(i > 0)
            def _():
                drain(rowsB, semB)
            fire(rowsA, idxA, semA)

            loadidx(k1, idxB)
            compute(k1, rowsB)
            drain(rowsA, semA)
            fire(rowsB, idxB, semB)
            return carry

        lax.fori_loop(0, _NCORN, pair, 0)
        drain(rowsB, semB)
        plsc.subcore_barrier()
        pltpu.sync_copy(sgrid.at[pl.ds(sid * rpt, rpt)],
                        out_hbm.at[cid, pl.ds(sid * rpt, rpt)])

    return k(U, idx3, w3)


def _slice(U, idx3, w3, gb):
    n = U.shape[0]
    mesh = plsc.VectorSubcoreMesh(core_axis_name="c", subcore_axis_name="s")

    @functools.partial(
        pl.kernel,
        mesh=mesh,
        compiler_params=_SC_PARAMS,
        out_type=jax.ShapeDtypeStruct((n, _DVAL), jnp.float32),
        scratch_types=[
            pltpu.VMEM((_SUBH, 128), jnp.int32),
            pltpu.VMEM((_SUBH, 128), jnp.int32),
            pltpu.VMEM((_SUBH, 128), jnp.float32),
            pltpu.VMEM((_PPT, _DVAL), jnp.float32),
            pltpu.VMEM((_HP, _DVAL), jnp.float32),
            pltpu.VMEM((_HP, _DVAL), jnp.float32),
            pltpu.SemaphoreType.DMA,
            pltpu.SemaphoreType.DMA,
        ],
    )
    def k(u_hbm, idx_hbm, w_hbm, g_hbm, out_hbm,
          idxA, idxB, wb, outacc, rowsA, rowsB, semA, semB):
        cid = lax.axis_index("c")
        sid = lax.axis_index("s")
        wid = sid * 2 + cid
        slab = pl.multiple_of(wid * _SUB, _SUB)

        pltpu.sync_copy(u_hbm.at[pl.ds(wid * _PPT, _PPT)], outacc)

        def neg(i, c2):
            outacc[i, :] = -outacc[i, :]
            return c2
        lax.fori_loop(0, _PPT, neg, 0)

        def fire(k, idxb, rows, sem):
            c = k // 2
            ho = k % 2
            pltpu.sync_copy(idx_hbm.at[c, pl.ds(slab + ho * _SUBH, _SUBH)], idxb)
            for b in range(_SUBH):
                pltpu.async_copy(g_hbm.at[idxb.at[b]],
                                 rows.at[pl.ds(b * 128, 128)], sem)

        def drain(rows, sem):
            pltpu.make_async_copy(g_hbm.at[pl.ds(0, _HP)], rows, sem).wait()

        def compute(k, rows):
            c = k // 2
            ho = k % 2
            pltpu.sync_copy(w_hbm.at[c, pl.ds(slab + ho * _SUBH, _SUBH)], wb)

            def grp(g, c2):
                r16 = g // 8
                col = (g % 8) * 16
                wv = wb[r16, pl.ds(col, 16)]
                base = r16 * 128 + col
                ob = ho * _HP + base
                for t in range(16):
                    outacc[ob + t, :] = (outacc[ob + t, :] +
                                         _bcast_lane(wv, t) * rows[base + t, :])
                return c2
            lax.fori_loop(0, _HP // 16, grp, 0)

        fire(0, idxA, rowsA, semA)
        nch = 2 * _NCORN

        def pair(i, carry):
            k0 = 2 * i
            k1 = 2 * i + 1
            fire(k1, idxB, rowsB, semB)
            drain(rowsA, semA)
            compute(k0, rowsA)

            @pl.when(i < _NCORN - 1)
            def _():
                fire(k0 + 2, idxA, rowsA, semA)
            drain(rowsB, semB)
            compute(k1, rowsB)
            return carry

        lax.fori_loop(0, nch // 2, pair, 0)
        pltpu.sync_copy(outacc, out_hbm.at[pl.ds(wid * _PPT, _PPT)])

    return k(U, idx3, w3, gb)


def kernel(U, ref):
    n = U.shape[0]
    ref3 = ref.T.reshape(_DREF, n // 128, 128)
    idx3, w3 = _corners(ref3)
    pg = _splat(U, idx3, w3)
    gb = _blur(pg)
    return _slice(U, idx3, w3, gb)


# speedup vs baseline: 1.5599x; 1.5599x over previous
import functools

import jax
import jax.numpy as jnp
from jax import lax
from jax.experimental import pallas as pl
from jax.experimental.pallas import tpu as pltpu
from jax.experimental.pallas import tpu_sc as plsc

_BINS = 8
_DREF = 5
_DVAL = 16
_G = _BINS ** _DREF
_NCORN = 1 << _DREF
_STRIDES = [_BINS ** (_DREF - 1 - d) for d in range(_DREF)]

_NB = 2048

_NTILES = 32
_CP = 64
_SUB = _CP * _NCORN // 128

_SC_PARAMS = pltpu.CompilerParams(needs_layout_passes=False,
                                  use_tc_tiling_on_sc=False)


def _corners_body(ref_ref, idx_ref, w_ref):
    r = ref_ref[...]
    scaled = r * (_BINS - 1)
    lo = jnp.floor(scaled)
    frac = scaled - lo
    lo_i = lo.astype(jnp.int32)
    cid = lax.broadcasted_iota(jnp.int32, (_NCORN, _NB), 0)
    w = jnp.ones((_NCORN, _NB), jnp.float32)
    idx = jnp.zeros((_NCORN, _NB), jnp.int32)
    for d in range(_DREF):
        bit = (cid >> d) & 1
        fb = jnp.broadcast_to(frac[d:d + 1, :], (_NCORN, _NB))
        lb = jnp.broadcast_to(lo_i[d:d + 1, :], (_NCORN, _NB))
        w = w * jnp.where(bit == 1, fb, 1.0 - fb)
        idx = idx + jnp.clip(lb + bit, 0, _BINS - 1) * _STRIDES[d]
    idx_ref[...] = idx
    w_ref[...] = w


def _corners(refT):
    n = refT.shape[1]
    return pl.pallas_call(
        _corners_body,
        grid=(n // _NB,),
        in_specs=[pl.BlockSpec((_DREF, _NB), lambda i: (0, i))],
        out_specs=[pl.BlockSpec((_NCORN, _NB), lambda i: (0, i)),
                   pl.BlockSpec((_NCORN, _NB), lambda i: (0, i))],
        out_shape=[jax.ShapeDtypeStruct((_NCORN, n), jnp.int32),
                   jax.ShapeDtypeStruct((_NCORN, n), jnp.float32)],
    )(refT)


def _blur_body(g_ref, out_ref):
    g = g_ref[0:_DVAL, :] + g_ref[_DVAL:2 * _DVAL, :]
    lane = lax.broadcasted_iota(jnp.int32, (_DVAL, _G), 1)
    for d in range(_DREF):
        s = _STRIDES[d]
        coord = (lane // s) % _BINS
        zs = jnp.zeros((_DVAL, s), jnp.float32)
        gl = jnp.concatenate([zs, g[:, : _G - s]], axis=1)
        gr = jnp.concatenate([g[:, s:], zs], axis=1)
        g = 0.5 * g + 0.25 * (jnp.where(coord > 0, gl, 0.0) +
                              jnp.where(coord < _BINS - 1, gr, 0.0))
    out_ref[...] = g


def _blur(gt):
    return pl.pallas_call(
        _blur_body,
        out_shape=jax.ShapeDtypeStruct((_DVAL, _G), jnp.float32),
    )(gt)


def _bcast_lane(vec, c):
    return jnp.take_along_axis(vec, jnp.full((16,), c, jnp.int32), axis=0)


def _load_point_chunk(u_hbm, w_hbm, off, ub, wb):
    pltpu.sync_copy(u_hbm.at[pl.ds(off, _CP)], ub)
    pltpu.sync_copy(w_hbm.at[pl.ds(pl.multiple_of(off * _NCORN, 2048),
                                   _CP * _NCORN)], wb)


def _load_idx_chunk(idx_hbm, off, idxb):
    pltpu.sync_copy(idx_hbm.at[pl.ds(pl.multiple_of(off * _NCORN // 128, 16),
                                     _SUB)], idxb)


def _splat(U, idx2, wflat, zg):
    n = U.shape[0]
    ppt = n // _NTILES
    nchunk = ppt // _CP
    mesh = plsc.VectorSubcoreMesh(core_axis_name="c", subcore_axis_name="s")

    @functools.partial(
        pl.kernel,
        mesh=mesh,
        compiler_params=_SC_PARAMS,
        out_type=jax.ShapeDtypeStruct((2, _G, _DVAL), jnp.float32),
        scratch_types=[
            pltpu.VMEM((_SUB, 128), jnp.int32),
            pltpu.VMEM((_SUB, 128), jnp.int32),
            pltpu.VMEM((_CP * _NCORN,), jnp.float32),
            pltpu.VMEM((_CP, _DVAL), jnp.float32),
            pltpu.VMEM((_CP * _NCORN, _DVAL), jnp.float32),
            pltpu.VMEM((_CP * _NCORN, _DVAL), jnp.float32),
            pltpu.VMEM_SHARED((_G, _DVAL), jnp.float32),
            pltpu.SemaphoreType.DMA,
            pltpu.SemaphoreType.DMA,
        ],
    )
    def k(u_hbm, idx_hbm, w_hbm, z_hbm, out_hbm,
          idxA, idxB, wb, ub, rowsA, rowsB, sgrid, semA, semB):
        cid = lax.axis_index("c")
        sid = lax.axis_index("s")
        wid = sid * 2 + cid
        rpt = _G // 16
        pltpu.sync_copy(z_hbm.at[pl.ds(sid * rpt, rpt)],
                        sgrid.at[pl.ds(sid * rpt, rpt)])
        plsc.subcore_barrier()
        base = wid * ppt

        def compute(off, rows):
            _load_point_chunk(u_hbm, w_hbm, off, ub, wb)

            def point(p, c2):
                u = ub[p, :]
                pb = pl.multiple_of(p * _NCORN, _NCORN)
                wv0 = wb[pl.ds(pb, 16)]
                wv1 = wb[pl.ds(pb + 16, 16)]
                for c in range(_NCORN):
                    wl = _bcast_lane(wv0 if c < 16 else wv1, c % 16)
                    rows[pb + c, :] = wl * u
                return c2
            lax.fori_loop(0, _CP, point, 0)

        def fire(rows, idxb, sem):
            for b in range(_SUB):
                pltpu.async_copy(rows.at[pl.ds(b * 128, 128)],
                                 sgrid.at[idxb.at[b]], sem, add=True)

        def drain(rows, sem):
            pltpu.make_async_copy(u_hbm.at[pl.ds(0, _CP * _NCORN)],
                                  rows, sem).wait()

        def pair(i, carry):
            off0 = pl.multiple_of(base + (2 * i) * _CP, _CP)
            off1 = pl.multiple_of(base + (2 * i + 1) * _CP, _CP)
            _load_idx_chunk(idx_hbm, off0, idxA)
            compute(off0, rowsA)

            @pl.when(i > 0)
            def _():
                drain(rowsB, semB)
            fire(rowsA, idxA, semA)

            _load_idx_chunk(idx_hbm, off1, idxB)
            compute(off1, rowsB)
            drain(rowsA, semA)
            fire(rowsB, idxB, semB)
            return carry

        lax.fori_loop(0, nchunk // 2, pair, 0)
        drain(rowsB, semB)
        plsc.subcore_barrier()
        pltpu.sync_copy(sgrid.at[pl.ds(sid * rpt, rpt)],
                        out_hbm.at[cid, pl.ds(sid * rpt, rpt)])

    return k(U, idx2, wflat, zg)


def _slice(U, idx2, wflat, gb):
    n = U.shape[0]
    ppt = n // _NTILES
    nchunk = ppt // _CP
    mesh = plsc.VectorSubcoreMesh(core_axis_name="c", subcore_axis_name="s")

    @functools.partial(
        pl.kernel,
        mesh=mesh,
        compiler_params=_SC_PARAMS,
        out_type=jax.ShapeDtypeStruct((n, _DVAL), jnp.float32),
        scratch_types=[
            pltpu.VMEM((_SUB, 128), jnp.int32),
            pltpu.VMEM((_SUB, 128), jnp.int32),
            pltpu.VMEM((_CP * _NCORN,), jnp.float32),
            pltpu.VMEM((_CP, _DVAL), jnp.float32),
            pltpu.VMEM((_CP * _NCORN, _DVAL), jnp.float32),
            pltpu.VMEM((_CP * _NCORN, _DVAL), jnp.float32),
            pltpu.VMEM((_CP, _DVAL), jnp.float32),
            pltpu.SemaphoreType.DMA,
            pltpu.SemaphoreType.DMA,
        ],
    )
    def k(u_hbm, idx_hbm, w_hbm, g_hbm, out_hbm,
          idxA, idxB, wb, ub, rowsA, rowsB, outb, semA, semB):
        cid = lax.axis_index("c")
        sid = lax.axis_index("s")
        wid = sid * 2 + cid
        base = wid * ppt

        def fire(off, idxb, rows, sem):
            _load_idx_chunk(idx_hbm, off, idxb)
            for b in range(_SUB):
                pltpu.async_copy(g_hbm.at[idxb.at[b]],
                                 rows.at[pl.ds(b * 128, 128)], sem)

        def drain(rows, sem):
            pltpu.make_async_copy(g_hbm.at[pl.ds(0, _CP * _NCORN)],
                                  rows, sem).wait()

        def compute(off, rows):
            _load_point_chunk(u_hbm, w_hbm, off, ub, wb)

            def point(p, c2):
                pb = pl.multiple_of(p * _NCORN, _NCORN)
                wv0 = wb[pl.ds(pb, 16)]
                wv1 = wb[pl.ds(pb + 16, 16)]
                acc = [-ub[p, :], jnp.zeros((16,), jnp.float32),
                       jnp.zeros((16,), jnp.float32), jnp.zeros((16,), jnp.float32)]
                for c in range(_NCORN):
                    wl = _bcast_lane(wv0 if c < 16 else wv1, c % 16)
                    acc[c % 4] = acc[c % 4] + wl * rows[pb + c, :]
                outb[p, :] = (acc[0] + acc[1]) + (acc[2] + acc[3])
                return c2
            lax.fori_loop(0, _CP, point, 0)
            pltpu.sync_copy(outb, out_hbm.at[pl.ds(off, _CP)])

        fire(base, idxA, rowsA, semA)

        def pair(i, carry):
            off0 = pl.multiple_of(base + (2 * i) * _CP, _CP)
            off1 = pl.multiple_of(base + (2 * i + 1) * _CP, _CP)
            off2 = pl.multiple_of(base + (2 * i + 2) * _CP, _CP)
            fire(off1, idxB, rowsB, semB)
            drain(rowsA, semA)
            compute(off0, rowsA)

            @pl.when(i < nchunk // 2 - 1)
            def _():
                fire(off2, idxA, rowsA, semA)
            drain(rowsB, semB)
            compute(off1, rowsB)
            return carry

        lax.fori_loop(0, nchunk // 2, pair, 0)

    return k(U, idx2, wflat, gb)


def kernel(U, ref):
    n = U.shape[0]
    refT = ref.T
    idxT, wT = _corners(refT)
    idx_flat = jnp.transpose(idxT).reshape(n * _NCORN)
    w_flat = jnp.transpose(wT).reshape(n * _NCORN)
    idx2 = idx_flat.reshape(n * _NCORN // 128, 128)
    zg = jnp.zeros((_G, _DVAL), jnp.float32)
    pg = _splat(U, idx2, w_flat, zg)
    gt = jnp.transpose(pg, (0, 2, 1)).reshape(2 * _DVAL, _G)
    gbT = _blur(gt)
    gb = jnp.transpose(gbT)
    return _slice(U, idx2, w_flat, gb)
